# baseline (speedup 1.0000x reference)
import jax
import jax.numpy as jnp
from jax import lax
from jax.experimental import pallas as pl
from jax.experimental.pallas import tpu as pltpu
from jax.experimental.pallas import tpu_sc as plsc

_N = 256
_B = 1024
_OUT = _N * (_N + 1) // 2
_NC = 2
_NS = 16
_NW = _NC * _NS
_BPW = _B // _NW
_PAD = 16

_OFF = [i * _N - (i * (i - 1)) // 2 for i in range(_N)]


def _body(x_hbm, out_hbm, inp, outbuf):
    wid = lax.axis_index("s") * _NC + lax.axis_index("c")

    def step(j, carry):
        b = wid * _BPW + j
        pltpu.sync_copy(x_hbm.at[b], inp.at[pl.ds(0, _N * _N)])
        for i in range(_N):
            seg_len = _N - i
            src = i * _N + i
            dst = _OFF[i]
            for t in range(0, seg_len, 16):
                outbuf[pl.ds(dst + t, 16)] = inp[pl.ds(src + t, 16)]
        pltpu.sync_copy(outbuf.at[pl.ds(0, _OUT)], out_hbm.at[b])
        return carry

    lax.fori_loop(0, _BPW, step, 0)


@jax.jit
def _run(x):
    f = pl.kernel(
        _body,
        out_type=jax.ShapeDtypeStruct((_B, _OUT), jnp.float32),
        mesh=plsc.VectorSubcoreMesh(core_axis_name="c", subcore_axis_name="s"),
        scratch_types=[
            pltpu.VMEM((_N * _N + _PAD,), jnp.float32),
            pltpu.VMEM((_OUT + _PAD,), jnp.float32),
        ],
    )
    return f(x)


def kernel(input):
    return _run(input.reshape(_B, _N * _N))


# speedup vs baseline: 1.8857x; 1.8857x over previous
import jax
import jax.numpy as jnp
from jax import lax
from jax.experimental import pallas as pl
from jax.experimental.pallas import tpu as pltpu
from jax.experimental.pallas import tpu_sc as plsc

_N = 256
_H = 128
_B = 1024
_OUT = _N * (_N + 1) // 2
_NC = 2
_NS = 16
_NW = _NC * _NS
_BPW = _B // _NW

_OFF = [i * _N - (i * (i - 1)) // 2 for i in range(_N)]


def _copy_rows(outbuf, stage, lo, hi, col0):
    for i in range(hi - 1, lo - 1, -1):
        seg_len = _N - i
        nch = (seg_len + 15) // 16
        for k in range(1, nch + 1):
            col = _N - 16 * k
            dst = _OFF[i] + seg_len - 16 * k
            outbuf[pl.ds(dst, 16)] = stage[i - lo, pl.ds(col - col0, 16)]


def _body(x_hbm, out_hbm, stage_lo, stage_hi, outbuf, sem_lo, sem_hi, sem_out):
    wid = lax.axis_index("s") * _NC + lax.axis_index("c")

    def step(j, carry):
        b = wid * _BPW + j
        cp_hi = pltpu.async_copy(
            x_hbm.at[b, pl.ds(_H, _H), pl.ds(_H, _H)], stage_hi, sem_hi)
        cp_lo = pltpu.async_copy(
            x_hbm.at[b, pl.ds(0, _H), pl.ds(0, _N)], stage_lo, sem_lo)
        cp_hi.wait()
        _copy_rows(outbuf, stage_hi, _H, _N, _H)
        cp_lo.wait()
        _copy_rows(outbuf, stage_lo, 0, _H, 0)
        pltpu.async_copy(outbuf, out_hbm.at[b], sem_out).wait()
        return carry

    lax.fori_loop(0, _BPW, step, 0)


@jax.jit
def _run(x):
    f = pl.kernel(
        _body,
        out_type=jax.ShapeDtypeStruct((_B, _OUT), jnp.float32),
        mesh=plsc.VectorSubcoreMesh(core_axis_name="c", subcore_axis_name="s"),
        scratch_types=[
            pltpu.VMEM((_H, _N), jnp.float32),
            pltpu.VMEM((_H, _H), jnp.float32),
            pltpu.VMEM((_OUT,), jnp.float32),
            pltpu.SemaphoreType.DMA,
            pltpu.SemaphoreType.DMA,
            pltpu.SemaphoreType.DMA,
        ],
    )
    return f(x)


def kernel(input):
    return _run(input)
